# baseline (speedup 1.0000x reference)
import jax
import jax.numpy as jnp
from jax.experimental import pallas as pl
from jax.experimental.pallas import tpu as pltpu

_THR = 0.8
_EPS = 1e-5


def _gcn_body(g_ref, wadj_ref, ga_ref, ba_ref, wwg_ref, gw_ref, bw_ref, out_ref):
    wadj = wadj_ref[:]
    g0 = g_ref[0]
    g1 = g_ref[1]
    z0 = jnp.dot(wadj, g0, preferred_element_type=jnp.float32)
    z1 = jnp.dot(wadj, g1, preferred_element_type=jnp.float32)
    n1 = 2.0 * g0.shape[1]
    mean = (jnp.sum(z0, axis=1) + jnp.sum(z1, axis=1)) / n1
    msq = (jnp.sum(z0 * z0, axis=1) + jnp.sum(z1 * z1, axis=1)) / n1
    var = msq - mean * mean
    scale = ga_ref[:] * jax.lax.rsqrt(var + _EPS)
    shift = ba_ref[:] - mean * scale
    h0 = jnp.maximum(z0 * scale[:, None] + shift[:, None], 0.0) + g0
    h1 = jnp.maximum(z1 * scale[:, None] + shift[:, None], 0.0) + g1
    wwg_t = wwg_ref[:].T
    y0 = jnp.dot(h0, wwg_t, preferred_element_type=jnp.float32)
    y1 = jnp.dot(h1, wwg_t, preferred_element_type=jnp.float32)
    n2 = 2.0 * g0.shape[0]
    mean2 = (jnp.sum(y0, axis=0) + jnp.sum(y1, axis=0)) / n2
    msq2 = (jnp.sum(y0 * y0, axis=0) + jnp.sum(y1 * y1, axis=0)) / n2
    var2 = msq2 - mean2 * mean2
    scale2 = gw_ref[:] * jax.lax.rsqrt(var2 + _EPS)
    shift2 = bw_ref[:] - mean2 * scale2
    out_ref[0] = jnp.maximum(y0 * scale2[None, :] + shift2[None, :], 0.0)
    out_ref[1] = jnp.maximum(y1 * scale2[None, :] + shift2[None, :], 0.0)


def _gcn(g, w_adj, gamma_adj, beta_adj, w_wg, gamma_wg, beta_wg):
    b, p, c = g.shape
    return pl.pallas_call(
        _gcn_body,
        out_shape=jax.ShapeDtypeStruct((b, p, c), jnp.float32),
    )(g, w_adj, gamma_adj, beta_adj, w_wg, gamma_wg, beta_wg)


def kernel(x, edge, W_adj, gamma_adj, beta_adj, W_wg, gamma_wg, beta_wg):
    b, c, h, w = x.shape
    p = W_adj.shape[0]
    hw = h * w
    e = jnp.where(edge < _THR, 0.0, edge).reshape(b, hw)
    _, point_indices = jax.lax.top_k(e, p)
    flat_x = x.reshape(b, c, hw)
    idx_exp = jnp.broadcast_to(point_indices[:, None, :], (b, c, p))
    feats = jnp.take_along_axis(flat_x, idx_exp, axis=2)
    g = jnp.transpose(feats, (0, 2, 1))
    z = _gcn(g, W_adj, gamma_adj, beta_adj, W_wg, gamma_wg, beta_wg)
    z = jnp.transpose(z, (0, 2, 1))
    bi = jnp.arange(b)[:, None, None]
    ci = jnp.arange(c)[None, :, None]
    out_flat = flat_x.at[bi, ci, idx_exp].set(z)
    return out_flat.reshape(b, c, h, w)
